# Initial kernel scaffold; baseline (speedup 1.0000x reference)
import jax
import jax.numpy as jnp
from jax.experimental import pallas as pl


def kernel(node_features, node_repr, edge_index, edge_types, num_relations, weight, bias):
    raise NotImplementedError("write your pallas kernel here")


# speedup vs baseline: 11.1372x; 11.1372x over previous
import functools

import jax
import jax.numpy as jnp
from jax import lax
from jax.experimental import pallas as pl
from jax.experimental.pallas import tpu as pltpu
from jax.experimental.pallas import tpu_sc as plsc

C = 128
NUM_CORES = 2
NUM_SUBCORES = 16
NW = NUM_CORES * NUM_SUBCORES


def _transform_kernel(x_ref, w_ref, o_ref):
    o_ref[...] = lax.dot_general(
        x_ref[...], w_ref[0],
        dimension_numbers=(((1,), (1,)), ((), ())),
        preferred_element_type=jnp.float32,
    )


def _combine_kernel(p0_ref, p1_ref, b_ref, o_ref):
    o_ref[...] = p0_ref[0] + p1_ref[0] + b_ref[...]


def _sc_body(n_pad, n_nodes, per_core_chunks, per_sub_chunks, rows_per_sub,
             h_ref, src_ref, et_ref, dst_ref, z_ref, out_ref,
             srcv, etv, gv, dstv, rows, acc, sem):
    c = lax.axis_index("c")
    s = lax.axis_index("s")
    row0 = s * rows_per_sub
    pltpu.sync_copy(z_ref.at[pl.ds(row0, rows_per_sub)],
                    acc.at[pl.ds(row0, rows_per_sub)])
    plsc.subcore_barrier()

    base_chunk = c * per_core_chunks + s * per_sub_chunks

    @pl.loop(0, per_sub_chunks)
    def _(i):
        ch = base_chunk + i
        pltpu.sync_copy(src_ref.at[ch], srcv)
        pltpu.sync_copy(et_ref.at[ch], etv)
        pltpu.sync_copy(dst_ref.at[ch], dstv)
        for k in range(C // 16):
            sl = pl.ds(k * 16, 16)
            gv[sl] = etv[sl] * n_nodes + srcv[sl]
        pltpu.async_copy(h_ref.at[gv], rows, sem).wait()
        pltpu.sync_copy(rows, acc.at[dstv], add=True)

    plsc.subcore_barrier()
    pltpu.sync_copy(acc.at[pl.ds(row0, rows_per_sub)],
                    out_ref.at[c, pl.ds(row0, rows_per_sub)])


def kernel(node_features, node_repr, edge_index, edge_types, num_relations,
           weight, bias):
    del node_features, num_relations
    n = node_repr.shape[0]
    d_in = node_repr.shape[1]
    r = weight.shape[0]
    d_out = weight.shape[1]
    e = edge_types.shape[0]

    bn = 1000
    assert n % bn == 0
    h_all = pl.pallas_call(
        _transform_kernel,
        grid=(r, n // bn),
        in_specs=[
            pl.BlockSpec((bn, d_in), lambda ri, ni: (ni, 0)),
            pl.BlockSpec((1, d_out, d_in), lambda ri, ni: (ri, 0, 0)),
        ],
        out_specs=pl.BlockSpec((bn, d_out), lambda ri, ni: (ri * (n // bn) + ni, 0)),
        out_shape=jax.ShapeDtypeStruct((r * n, d_out), jnp.float32),
    )(node_repr, weight)

    chunks_total = -(-e // (C * NW)) * NW
    e_pad = chunks_total * C
    pad = e_pad - e
    src_p = jnp.concatenate(
        [edge_index[0], jnp.zeros((pad,), jnp.int32)]).reshape(chunks_total, C)
    et_p = jnp.concatenate(
        [edge_types, jnp.zeros((pad,), jnp.int32)]).reshape(chunks_total, C)
    dst_p = jnp.concatenate(
        [edge_index[1], jnp.full((pad,), n, jnp.int32)]).reshape(chunks_total, C)

    per_core_chunks = chunks_total // NUM_CORES
    per_sub_chunks = per_core_chunks // NUM_SUBCORES
    rows_per_sub = -(-(n + 1) // (NUM_SUBCORES * 8)) * 8
    n_pad = rows_per_sub * NUM_SUBCORES
    zeros_init = jnp.zeros((n_pad, d_out), jnp.float32)

    mesh = plsc.VectorSubcoreMesh(core_axis_name="c", subcore_axis_name="s")
    sc_kernel = pl.kernel(
        functools.partial(_sc_body, n_pad, n, per_core_chunks, per_sub_chunks,
                          rows_per_sub),
        out_type=jax.ShapeDtypeStruct((NUM_CORES, n_pad, d_out), jnp.float32),
        mesh=mesh,
        scratch_types=[
            pltpu.VMEM((C,), jnp.int32),
            pltpu.VMEM((C,), jnp.int32),
            pltpu.VMEM((C,), jnp.int32),
            pltpu.VMEM((C,), jnp.int32),
            pltpu.VMEM((C, 128), jnp.float32),
            pltpu.VMEM_SHARED((n_pad, 128), jnp.float32),
            pltpu.SemaphoreType.DMA,
        ],
    )
    partials = sc_kernel(h_all, src_p, et_p, dst_p, zeros_init)

    out = pl.pallas_call(
        _combine_kernel,
        grid=(n // bn,),
        in_specs=[
            pl.BlockSpec((1, bn, d_out), lambda i: (0, i, 0)),
            pl.BlockSpec((1, bn, d_out), lambda i: (1, i, 0)),
            pl.BlockSpec((1, d_out), lambda i: (0, 0)),
        ],
        out_specs=pl.BlockSpec((bn, d_out), lambda i: (i, 0)),
        out_shape=jax.ShapeDtypeStruct((n, d_out), jnp.float32),
    )(partials, partials, bias.reshape(1, d_out))

    return out
